# baseline (speedup 1.0000x reference)
import functools
import math

import jax
import jax.numpy as jnp
from jax.experimental import pallas as pl
from jax.experimental.pallas import tpu as pltpu

_EPS = 1e-05
_BETA = 0.7
_LOG2E = math.log2(math.e)
_LN2 = math.log(2.0)
_XLO = (math.log(_EPS) - math.log1p(-_EPS)) / _LN2
_XHI = -_XLO


def _loss_body(c_ref, x_ref, t_ref, out_ref, acc_ref, cnt_ref, *, bs, o, rows):
    i = pl.program_id(0)

    @pl.when(i == 0)
    def _init():
        acc_ref[...] = jnp.zeros_like(acc_ref)
        cnt_ref[0] = 0.0

    chunk = 512
    nch = x_ref.shape[0] // chunk
    for k in range(nch):
        rs = pl.ds(k * chunk, chunk)
        x = x_ref[rs, :]
        tgt = t_ref[rs, :]
        m_row = (c_ref[0, 0:1, rs] == 0).astype(jnp.float32)

        row_id = jax.lax.broadcasted_iota(jnp.int32, (8, chunk), 0)
        lhs = jnp.where(row_id == 0, 1.0, jnp.where(row_id == 1, m_row, 0.0))
        lhs_bf = lhs.astype(jnp.bfloat16)

        Xc = jnp.clip(x * _LOG2E, _XLO, _XHI)
        w = tgt * Xc
        e2 = jnp.exp2(jnp.minimum(Xc, -Xc))
        l2 = jnp.log2(1.0 + e2)
        lp2 = jnp.minimum(Xc, 0.0) - l2
        pred = jnp.exp2(lp2)
        base2 = (lp2 - Xc) + w
        q = pred * Xc - w
        acc_ref[0:8, :] += jnp.dot(lhs_bf, base2.astype(jnp.bfloat16),
                                   preferred_element_type=jnp.float32)
        acc_ref[8:16, :] += jnp.dot(lhs_bf, q.astype(jnp.bfloat16),
                                    preferred_element_type=jnp.float32)
        cnt_ref[0] += jnp.sum(m_row)

    @pl.when(i == pl.num_programs(0) - 1)
    def _finish():
        sum_base_all = jnp.sum(acc_ref[0:1, :])
        sum_base_msk = jnp.sum(acc_ref[1:2, :])
        sum_q_msk = jnp.sum(acc_ref[9:10, :])
        noisy_sum = -_LN2 * (sum_base_msk + (1.0 - _BETA) * sum_q_msk)
        cur_sum = -_LN2 * (sum_base_all - sum_base_msk)
        nl = cnt_ref[0]
        cl = float(rows) - nl
        noisy_loss = noisy_sum / (nl * float(o)) * (nl / float(bs))
        curated_loss = cur_sum / (cl * float(o)) * (cl / float(bs))
        out_ref[0] = noisy_loss * 0.5 + curated_loss * 0.5
        out_ref[1] = noisy_loss
        out_ref[2] = curated_loss


def kernel(output, target, clean):
    bs, seq, o = target.shape
    rows = bs * seq
    x = output.reshape(rows, o)
    t = target.reshape(rows, o)
    block_rows = 2048
    grid = rows // block_rows
    c = clean.reshape(grid, 1, block_rows)
    body = functools.partial(_loss_body, bs=bs, o=o, rows=rows)
    out = pl.pallas_call(
        body,
        grid=(grid,),
        in_specs=[
            pl.BlockSpec((1, 1, block_rows), lambda i: (i, 0, 0)),
            pl.BlockSpec((block_rows, o), lambda i: (i, 0)),
            pl.BlockSpec((block_rows, o), lambda i: (i, 0)),
        ],
        out_specs=pl.BlockSpec(memory_space=pltpu.SMEM),
        out_shape=jax.ShapeDtypeStruct((3,), jnp.float32),
        scratch_shapes=[
            pltpu.VMEM((24, o), jnp.float32),
            pltpu.SMEM((1,), jnp.float32),
        ],
    )(c, x, t)
    return (out[0], out[1], out[2])


# speedup vs baseline: 1.0609x; 1.0609x over previous
import functools
import math

import jax
import jax.numpy as jnp
from jax.experimental import pallas as pl
from jax.experimental.pallas import tpu as pltpu

_EPS = 1e-05
_BETA = 0.7
_LOG2E = math.log2(math.e)
_LN2 = math.log(2.0)
_XLO = (math.log(_EPS) - math.log1p(-_EPS)) / _LN2
_XHI = -_XLO


def _loss_body(c_ref, x_ref, t_ref, out_ref, acc_ref, cnt_ref, *, bs, o, rows):
    i = pl.program_id(0)

    @pl.when(i == 0)
    def _init():
        acc_ref[...] = jnp.zeros_like(acc_ref)
        cnt_ref[0] = 0.0

    chunk = 1024
    nch = x_ref.shape[0] // chunk
    for k in range(nch):
        rs = pl.ds(k * chunk, chunk)
        x = x_ref[rs, :]
        tgt = t_ref[rs, :]
        m_row = (c_ref[0, 0:1, rs] == 0).astype(jnp.float32)

        row_id = jax.lax.broadcasted_iota(jnp.int32, (8, chunk), 0)
        lhs = jnp.where(row_id == 0, 1.0, jnp.where(row_id == 1, m_row, 0.0))
        lhs_bf = lhs.astype(jnp.bfloat16)

        Xc = jnp.clip(x * _LOG2E, _XLO, _XHI)
        w = tgt * Xc
        e2 = jnp.exp2(jnp.minimum(Xc, -Xc))
        l2 = jnp.log2(1.0 + e2)
        lp2 = jnp.minimum(Xc, 0.0) - l2
        pred = jnp.exp2(lp2)
        base2 = (lp2 - Xc) + w
        q = pred * Xc - w
        acc_ref[0:8, :] += jnp.dot(lhs_bf, base2.astype(jnp.bfloat16),
                                   preferred_element_type=jnp.float32)
        acc_ref[8:16, :] += jnp.dot(lhs_bf, q.astype(jnp.bfloat16),
                                    preferred_element_type=jnp.float32)
        cnt_ref[0] += jnp.sum(m_row)

    @pl.when(i == pl.num_programs(0) - 1)
    def _finish():
        sum_base_all = jnp.sum(acc_ref[0:1, :])
        sum_base_msk = jnp.sum(acc_ref[1:2, :])
        sum_q_msk = jnp.sum(acc_ref[9:10, :])
        noisy_sum = -_LN2 * (sum_base_msk + (1.0 - _BETA) * sum_q_msk)
        cur_sum = -_LN2 * (sum_base_all - sum_base_msk)
        nl = cnt_ref[0]
        cl = float(rows) - nl
        noisy_loss = noisy_sum / (nl * float(o)) * (nl / float(bs))
        curated_loss = cur_sum / (cl * float(o)) * (cl / float(bs))
        out_ref[0] = noisy_loss * 0.5 + curated_loss * 0.5
        out_ref[1] = noisy_loss
        out_ref[2] = curated_loss


def kernel(output, target, clean):
    bs, seq, o = target.shape
    rows = bs * seq
    x = output.reshape(rows, o)
    t = target.reshape(rows, o)
    block_rows = 4096
    grid = rows // block_rows
    c = clean.reshape(grid, 1, block_rows)
    body = functools.partial(_loss_body, bs=bs, o=o, rows=rows)
    out = pl.pallas_call(
        body,
        grid=(grid,),
        in_specs=[
            pl.BlockSpec((1, 1, block_rows), lambda i: (i, 0, 0)),
            pl.BlockSpec((block_rows, o), lambda i: (i, 0)),
            pl.BlockSpec((block_rows, o), lambda i: (i, 0)),
        ],
        out_specs=pl.BlockSpec(memory_space=pltpu.SMEM),
        out_shape=jax.ShapeDtypeStruct((3,), jnp.float32),
        scratch_shapes=[
            pltpu.VMEM((16, o), jnp.float32),
            pltpu.SMEM((1,), jnp.float32),
        ],
    )(c, x, t)
    return (out[0], out[1], out[2])
